# baseline (speedup 1.0000x reference)
import functools

import jax
import jax.numpy as jnp
import numpy as np
from jax import lax
from jax.experimental import pallas as pl
from jax.experimental.pallas import tpu as pltpu
from jax.experimental.pallas import tpu_sc as plsc

VOCAB = 30000
EMB_DIM = 2
BATCH = 16384
HIST = 200
COLS_A = 128
COLS_B = HIST - COLS_A

NUM_CORES = 2
NUM_SUBCORES = 16
LANES = 16
NW = NUM_CORES * NUM_SUBCORES
ROWS_PER_W = BATCH // NW
CHUNK = 64
N_CHUNKS = ROWS_PER_W // CHUNK
GROUPS = CHUNK // LANES


def _mesh_kernel():
    mesh = plsc.VectorSubcoreMesh(core_axis_name="c", subcore_axis_name="s")

    @functools.partial(
        pl.kernel,
        mesh=mesh,
        compiler_params=pltpu.CompilerParams(needs_layout_passes=False),
        out_type=jax.ShapeDtypeStruct((BATCH * EMB_DIM,), jnp.float32),
        scratch_types=[
            pltpu.VMEM((VOCAB * EMB_DIM,), jnp.float32),
            pltpu.VMEM((VOCAB,), jnp.float32),
            pltpu.VMEM((CHUNK, COLS_A), jnp.int32),
            pltpu.VMEM((CHUNK, COLS_B), jnp.int32),
            pltpu.VMEM((CHUNK * EMB_DIM,), jnp.float32),
        ],
    )
    def body(x_hbm, emb_hbm, out_hbm, emb_v, diff_v, xa_v, xb_v, out_v):
        wid = lax.axis_index("s") * NUM_CORES + lax.axis_index("c")
        iota = lax.broadcasted_iota(jnp.int32, (LANES,), 0)
        zero_i = jnp.zeros((LANES,), jnp.int32)
        zero_f = jnp.zeros((LANES,), jnp.float32)

        pltpu.sync_copy(emb_hbm, emb_v)
        iota2 = iota * 2

        def build(j, carry):
            ridx = iota + j * LANES
            ridx2 = iota2 + j * (2 * LANES)
            v0 = plsc.load_gather(emb_v, [ridx2])
            v1 = plsc.load_gather(emb_v, [ridx2 + 1])
            plsc.store_scatter(diff_v, [ridx], v1 - v0)
            return carry

        lax.fori_loop(0, VOCAB // LANES, build, 0, unroll=5)
        plsc.store_scatter(diff_v, [zero_i], zero_f, mask=iota < 1)

        base_row = wid * ROWS_PER_W
        inv_hist = np.float32(1.0 / HIST)

        def chunk_body(c, chunk_carry):
            row0 = base_row + c * CHUNK
            pltpu.sync_copy(x_hbm.at[pl.ds(row0, CHUNK), pl.ds(0, COLS_A)],
                            xa_v)
            pltpu.sync_copy(
                x_hbm.at[pl.ds(row0, CHUNK), pl.ds(COLS_A, COLS_B)], xb_v)
            for g in range(GROUPS):
                rowg = iota + g * LANES

                def step_a(l, carry, rowg=rowg):
                    acc0, acc1, colv = carry
                    xv0 = plsc.load_gather(xa_v, [rowg, colv])
                    dv0 = plsc.load_gather(diff_v, [xv0])
                    xv1 = plsc.load_gather(xa_v, [rowg, colv + 1])
                    dv1 = plsc.load_gather(diff_v, [xv1])
                    return (acc0 + dv0, acc1 + dv1, colv + 2)

                def step_b(l, carry, rowg=rowg):
                    acc0, acc1, colv = carry
                    xv0 = plsc.load_gather(xb_v, [rowg, colv])
                    dv0 = plsc.load_gather(diff_v, [xv0])
                    xv1 = plsc.load_gather(xb_v, [rowg, colv + 1])
                    dv1 = plsc.load_gather(diff_v, [xv1])
                    return (acc0 + dv0, acc1 + dv1, colv + 2)

                acc0, acc1, _ = lax.fori_loop(
                    0, COLS_A // 2, step_a, (zero_f, zero_f, zero_i),
                    unroll=4)
                acc0, acc1, _ = lax.fori_loop(
                    0, COLS_B // 2, step_b, (acc0, acc1, zero_i), unroll=4)
                acc = acc0 + acc1
                d = acc * inv_hist
                e = jnp.exp(d)
                p0 = 1.0 / (1.0 + e)
                p1 = 1.0 - p0
                oidx = (iota + g * LANES) * EMB_DIM
                plsc.store_scatter(out_v, [oidx], p0)
                plsc.store_scatter(out_v, [oidx + 1], p1)
            pltpu.sync_copy(
                out_v, out_hbm.at[pl.ds(row0 * EMB_DIM, CHUNK * EMB_DIM)])
            return chunk_carry

        lax.fori_loop(0, N_CHUNKS, chunk_body, 0)

    return body


_sc_kernel = _mesh_kernel()


@jax.jit
def kernel(x, emb):
    out_flat = _sc_kernel(x, emb.reshape(-1))
    return out_flat.reshape(BATCH, EMB_DIM)


# speedup vs baseline: 3.2397x; 3.2397x over previous
import functools

import jax
import jax.numpy as jnp
import numpy as np
from jax import lax
from jax.experimental import pallas as pl
from jax.experimental.pallas import tpu as pltpu
from jax.experimental.pallas import tpu_sc as plsc

VOCAB = 30000
EMB_DIM = 2
BATCH = 16384
HIST = 200

NUM_CORES = 2
NUM_SUBCORES = 16
LANES = 16
NW = NUM_CORES * NUM_SUBCORES
ROWS_PER_W = BATCH // NW
BLOCK = 128
N_BLOCKS = ROWS_PER_W // BLOCK
GROUPS = BLOCK // LANES


def _mesh_kernel():
    mesh = plsc.VectorSubcoreMesh(core_axis_name="c", subcore_axis_name="s")

    @functools.partial(
        pl.kernel,
        mesh=mesh,
        compiler_params=pltpu.CompilerParams(needs_layout_passes=False),
        out_type=(
            jax.ShapeDtypeStruct((BATCH,), jnp.float32),
            jax.ShapeDtypeStruct((BATCH,), jnp.float32),
        ),
        scratch_types=[
            pltpu.VMEM((VOCAB,), jnp.float32),
            pltpu.VMEM((HIST, BLOCK), jnp.int32),
            pltpu.VMEM((BLOCK,), jnp.float32),
            pltpu.VMEM((BLOCK,), jnp.float32),
        ],
    )
    def body(xt_hbm, diff_hbm, p0_hbm, p1_hbm, diff_v, x_v, p0_v, p1_v):
        wid = lax.axis_index("s") * NUM_CORES + lax.axis_index("c")
        iota = lax.broadcasted_iota(jnp.int32, (LANES,), 0)
        zero_i = jnp.zeros((LANES,), jnp.int32)
        one_i = jnp.ones((LANES,), jnp.int32)
        zero_f = jnp.zeros((LANES,), jnp.float32)

        pltpu.sync_copy(diff_hbm, diff_v)
        plsc.store_scatter(diff_v, [zero_i], zero_f, mask=iota < 1)

        base_row = wid * ROWS_PER_W
        inv_hist = np.float32(1.0 / HIST)

        for c in range(N_BLOCKS):
            row0 = base_row + c * BLOCK
            pltpu.sync_copy(xt_hbm.at[pl.ds(0, HIST), pl.ds(row0, BLOCK)],
                            x_v)
            for g in range(GROUPS):
                bvec = iota + g * LANES

                def step(l, carry, bvec=bvec):
                    acc0, acc1, lvec = carry
                    xv0 = plsc.load_gather(x_v, [lvec, bvec])
                    dv0 = plsc.load_gather(diff_v, [xv0])
                    xv1 = plsc.load_gather(x_v, [lvec + one_i, bvec])
                    dv1 = plsc.load_gather(diff_v, [xv1])
                    return (acc0 + dv0, acc1 + dv1, lvec + 2)

                acc0, acc1, _ = lax.fori_loop(
                    0, HIST // 2, step, (zero_f, zero_f, zero_i), unroll=4)
                acc = acc0 + acc1
                d = acc * inv_hist
                e = jnp.exp(d)
                p0 = 1.0 / (1.0 + e)
                p1 = 1.0 - p0
                p0_v[pl.ds(g * LANES, LANES)] = p0
                p1_v[pl.ds(g * LANES, LANES)] = p1
            pltpu.sync_copy(p0_v, p0_hbm.at[pl.ds(row0, BLOCK)])
            pltpu.sync_copy(p1_v, p1_hbm.at[pl.ds(row0, BLOCK)])

    return body


_sc_kernel = _mesh_kernel()


@jax.jit
def kernel(x, emb):
    diff = emb[:, 1] - emb[:, 0]
    p0, p1 = _sc_kernel(x.T, diff)
    return jnp.stack([p0, p1], axis=1)


# speedup vs baseline: 3.6548x; 1.1281x over previous
import functools

import jax
import jax.numpy as jnp
import numpy as np
from jax import lax
from jax.experimental import pallas as pl
from jax.experimental.pallas import tpu as pltpu
from jax.experimental.pallas import tpu_sc as plsc

VOCAB = 30000
EMB_DIM = 2
BATCH = 16384
HIST = 200

NUM_CORES = 2
NUM_SUBCORES = 16
LANES = 16
NW = NUM_CORES * NUM_SUBCORES
ROWS_PER_W = BATCH // NW
BLOCK = 128
N_BLOCKS = ROWS_PER_W // BLOCK
GROUPS = BLOCK // LANES


def _mesh_kernel():
    mesh = plsc.VectorSubcoreMesh(core_axis_name="c", subcore_axis_name="s")

    @functools.partial(
        pl.kernel,
        mesh=mesh,
        compiler_params=pltpu.CompilerParams(needs_layout_passes=False),
        out_type=(
            jax.ShapeDtypeStruct((BATCH,), jnp.float32),
            jax.ShapeDtypeStruct((BATCH,), jnp.float32),
        ),
        scratch_types=[
            pltpu.VMEM((VOCAB,), jnp.float32),
            pltpu.VMEM((HIST, BLOCK), jnp.int32),
            pltpu.VMEM((HIST, BLOCK), jnp.int32),
            pltpu.VMEM((BLOCK,), jnp.float32),
            pltpu.VMEM((BLOCK,), jnp.float32),
            pltpu.SemaphoreType.DMA,
            pltpu.SemaphoreType.DMA,
            pltpu.SemaphoreType.DMA,
        ],
    )
    def body(xt_hbm, diff_hbm, p0_hbm, p1_hbm,
             diff_v, xa_v, xb_v, p0_v, p1_v, sem_d, sem_a, sem_b):
        wid = lax.axis_index("s") * NUM_CORES + lax.axis_index("c")
        iota = lax.broadcasted_iota(jnp.int32, (LANES,), 0)
        zero_i = jnp.zeros((LANES,), jnp.int32)
        one_i = jnp.ones((LANES,), jnp.int32)
        zero_f = jnp.zeros((LANES,), jnp.float32)

        base_row = wid * ROWS_PER_W
        inv_hist = np.float32(1.0 / HIST)

        bufs = (xa_v, xb_v)
        sems = (sem_a, sem_b)

        def start_block(c):
            row0 = base_row + c * BLOCK
            return pltpu.async_copy(
                xt_hbm.at[pl.ds(0, HIST), pl.ds(row0, BLOCK)],
                bufs[c % 2], sems[c % 2])

        diff_cp = pltpu.async_copy(diff_hbm, diff_v, sem_d)
        pending = start_block(0)
        diff_cp.wait()
        plsc.store_scatter(diff_v, [zero_i], zero_f, mask=iota < 1)

        for c in range(N_BLOCKS):
            row0 = base_row + c * BLOCK
            x_v = bufs[c % 2]
            pending.wait()
            if c + 1 < N_BLOCKS:
                pending = start_block(c + 1)
            for g in range(GROUPS):
                bvec = iota + g * LANES

                def step(l, carry, bvec=bvec):
                    acc0, acc1, lvec = carry
                    xv0 = plsc.load_gather(x_v, [lvec, bvec])
                    dv0 = plsc.load_gather(diff_v, [xv0])
                    xv1 = plsc.load_gather(x_v, [lvec + one_i, bvec])
                    dv1 = plsc.load_gather(diff_v, [xv1])
                    return (acc0 + dv0, acc1 + dv1, lvec + 2)

                acc0, acc1, _ = lax.fori_loop(
                    0, HIST // 2, step, (zero_f, zero_f, zero_i), unroll=4)
                acc = acc0 + acc1
                d = acc * inv_hist
                e = jnp.exp(d)
                p0 = 1.0 / (1.0 + e)
                p1 = 1.0 - p0
                p0_v[pl.ds(g * LANES, LANES)] = p0
                p1_v[pl.ds(g * LANES, LANES)] = p1
            pltpu.sync_copy(p0_v, p0_hbm.at[pl.ds(row0, BLOCK)])
            pltpu.sync_copy(p1_v, p1_hbm.at[pl.ds(row0, BLOCK)])

    return body


_sc_kernel = _mesh_kernel()


@jax.jit
def kernel(x, emb):
    diff = emb[:, 1] - emb[:, 0]
    p0, p1 = _sc_kernel(x.T, diff)
    return jnp.stack([p0, p1], axis=1)


# speedup vs baseline: 3.9542x; 1.0819x over previous
import functools

import jax
import jax.numpy as jnp
import numpy as np
from jax import lax
from jax.experimental import pallas as pl
from jax.experimental.pallas import tpu as pltpu
from jax.experimental.pallas import tpu_sc as plsc

VOCAB = 30000
EMB_DIM = 2
BATCH = 16384
HIST = 200

NUM_CORES = 2
NUM_SUBCORES = 16
LANES = 16
NW = NUM_CORES * NUM_SUBCORES
ROWS_PER_W = BATCH // NW
BLOCK = 128
N_BLOCKS = ROWS_PER_W // BLOCK
GROUPS = BLOCK // LANES


def _mesh_kernel():
    mesh = plsc.VectorSubcoreMesh(core_axis_name="c", subcore_axis_name="s")

    @functools.partial(
        pl.kernel,
        mesh=mesh,
        compiler_params=pltpu.CompilerParams(needs_layout_passes=False),
        out_type=(
            jax.ShapeDtypeStruct((BATCH,), jnp.float32),
            jax.ShapeDtypeStruct((BATCH,), jnp.float32),
        ),
        scratch_types=[
            pltpu.VMEM((VOCAB,), jnp.float32),
            pltpu.VMEM((HIST, BLOCK), jnp.int32),
            pltpu.VMEM((HIST, BLOCK), jnp.int32),
            pltpu.VMEM((BLOCK,), jnp.float32),
            pltpu.VMEM((BLOCK,), jnp.float32),
            pltpu.SemaphoreType.DMA,
            pltpu.SemaphoreType.DMA,
            pltpu.SemaphoreType.DMA,
        ],
    )
    def body(xt_hbm, diff_hbm, p0_hbm, p1_hbm,
             diff_v, xa_v, xb_v, p0_v, p1_v, sem_d, sem_a, sem_b):
        wid = lax.axis_index("s") * NUM_CORES + lax.axis_index("c")
        iota = lax.broadcasted_iota(jnp.int32, (LANES,), 0)
        zero_i = jnp.zeros((LANES,), jnp.int32)
        one_i = jnp.ones((LANES,), jnp.int32)
        zero_f = jnp.zeros((LANES,), jnp.float32)

        base_row = wid * ROWS_PER_W
        inv_hist = np.float32(1.0 / HIST)

        bufs = (xa_v, xb_v)
        sems = (sem_a, sem_b)

        def start_block(c):
            row0 = base_row + c * BLOCK
            return pltpu.async_copy(
                xt_hbm.at[pl.ds(0, HIST), pl.ds(row0, BLOCK)],
                bufs[c % 2], sems[c % 2])

        diff_cp = pltpu.async_copy(diff_hbm, diff_v, sem_d)
        pending = start_block(0)
        diff_cp.wait()
        plsc.store_scatter(diff_v, [zero_i], zero_f, mask=iota < 1)

        for c in range(N_BLOCKS):
            row0 = base_row + c * BLOCK
            x_v = bufs[c % 2]
            pending.wait()
            if c + 1 < N_BLOCKS:
                pending = start_block(c + 1)
            def group_body(g, group_carry, x_v=x_v):
                bvec = iota + g * LANES

                def step(l, carry):
                    acc0, acc1, lvec = carry
                    xv0 = plsc.load_gather(x_v, [lvec, bvec])
                    dv0 = plsc.load_gather(diff_v, [xv0])
                    xv1 = plsc.load_gather(x_v, [lvec + one_i, bvec])
                    dv1 = plsc.load_gather(diff_v, [xv1])
                    return (acc0 + dv0, acc1 + dv1, lvec + 2)

                acc0, acc1, _ = lax.fori_loop(
                    0, HIST // 2, step, (zero_f, zero_f, zero_i), unroll=10)
                acc = acc0 + acc1
                d = acc * inv_hist
                e = jnp.exp(d)
                p0 = 1.0 / (1.0 + e)
                p1 = 1.0 - p0
                p0_v[pl.ds(g * LANES, LANES)] = p0
                p1_v[pl.ds(g * LANES, LANES)] = p1
                return group_carry

            lax.fori_loop(0, GROUPS, group_body, 0)
            pltpu.sync_copy(p0_v, p0_hbm.at[pl.ds(row0, BLOCK)])
            pltpu.sync_copy(p1_v, p1_hbm.at[pl.ds(row0, BLOCK)])

    return body


_sc_kernel = _mesh_kernel()


@jax.jit
def kernel(x, emb):
    diff = emb[:, 1] - emb[:, 0]
    p0, p1 = _sc_kernel(x.T, diff)
    return jnp.stack([p0, p1], axis=1)


# speedup vs baseline: 3.9760x; 1.0055x over previous
import functools

import jax
import jax.numpy as jnp
import numpy as np
from jax import lax
from jax.experimental import pallas as pl
from jax.experimental.pallas import tpu as pltpu
from jax.experimental.pallas import tpu_sc as plsc

VOCAB = 30000
EMB_DIM = 2
BATCH = 16384
HIST = 200

NUM_CORES = 2
NUM_SUBCORES = 16
LANES = 16
NW = NUM_CORES * NUM_SUBCORES
ROWS_PER_W = BATCH // NW
BLOCK = 128
N_BLOCKS = ROWS_PER_W // BLOCK
GROUPS = BLOCK // LANES


def _mesh_kernel():
    mesh = plsc.VectorSubcoreMesh(core_axis_name="c", subcore_axis_name="s")

    @functools.partial(
        pl.kernel,
        mesh=mesh,
        compiler_params=pltpu.CompilerParams(needs_layout_passes=False),
        out_type=(
            jax.ShapeDtypeStruct((BATCH,), jnp.float32),
            jax.ShapeDtypeStruct((BATCH,), jnp.float32),
        ),
        scratch_types=[
            pltpu.VMEM((VOCAB,), jnp.float32),
            pltpu.VMEM((HIST, BLOCK), jnp.int32),
            pltpu.VMEM((HIST, BLOCK), jnp.int32),
            pltpu.VMEM((BLOCK,), jnp.float32),
            pltpu.VMEM((BLOCK,), jnp.float32),
            pltpu.SemaphoreType.DMA,
            pltpu.SemaphoreType.DMA,
            pltpu.SemaphoreType.DMA,
        ],
    )
    def body(xt_hbm, diff_hbm, p0_hbm, p1_hbm,
             diff_v, xa_v, xb_v, p0_v, p1_v, sem_d, sem_a, sem_b):
        wid = lax.axis_index("s") * NUM_CORES + lax.axis_index("c")
        iota = lax.broadcasted_iota(jnp.int32, (LANES,), 0)
        zero_i = jnp.zeros((LANES,), jnp.int32)
        one_i = jnp.ones((LANES,), jnp.int32)
        zero_f = jnp.zeros((LANES,), jnp.float32)

        base_row = wid * ROWS_PER_W
        inv_hist = np.float32(1.0 / HIST)

        bufs = (xa_v, xb_v)
        sems = (sem_a, sem_b)

        def start_block(c):
            row0 = base_row + c * BLOCK
            return pltpu.async_copy(
                xt_hbm.at[pl.ds(0, HIST), pl.ds(row0, BLOCK)],
                bufs[c % 2], sems[c % 2])

        diff_cp = pltpu.async_copy(diff_hbm, diff_v, sem_d)
        pending = start_block(0)
        diff_cp.wait()
        plsc.store_scatter(diff_v, [zero_i], zero_f, mask=iota < 1)

        for c in range(N_BLOCKS):
            row0 = base_row + c * BLOCK
            x_v = bufs[c % 2]
            pending.wait()
            if c + 1 < N_BLOCKS:
                pending = start_block(c + 1)
            def group_body(g, group_carry, x_v=x_v):
                bvec = iota + g * LANES

                def step(l, carry):
                    acc0, acc1, acc2, acc3, lvec = carry
                    xv0 = plsc.load_gather(x_v, [lvec, bvec])
                    dv0 = plsc.load_gather(diff_v, [xv0])
                    xv1 = plsc.load_gather(x_v, [lvec + 1, bvec])
                    dv1 = plsc.load_gather(diff_v, [xv1])
                    xv2 = plsc.load_gather(x_v, [lvec + 2, bvec])
                    dv2 = plsc.load_gather(diff_v, [xv2])
                    xv3 = plsc.load_gather(x_v, [lvec + 3, bvec])
                    dv3 = plsc.load_gather(diff_v, [xv3])
                    return (acc0 + dv0, acc1 + dv1, acc2 + dv2, acc3 + dv3,
                            lvec + 4)

                acc0, acc1, acc2, acc3, _ = lax.fori_loop(
                    0, HIST // 4, step,
                    (zero_f, zero_f, zero_f, zero_f, zero_i), unroll=5)
                acc = (acc0 + acc1) + (acc2 + acc3)
                d = acc * inv_hist
                e = jnp.exp(d)
                p0 = 1.0 / (1.0 + e)
                p1 = 1.0 - p0
                p0_v[pl.ds(g * LANES, LANES)] = p0
                p1_v[pl.ds(g * LANES, LANES)] = p1
                return group_carry

            lax.fori_loop(0, GROUPS, group_body, 0)
            pltpu.sync_copy(p0_v, p0_hbm.at[pl.ds(row0, BLOCK)])
            pltpu.sync_copy(p1_v, p1_hbm.at[pl.ds(row0, BLOCK)])

    return body


_sc_kernel = _mesh_kernel()


@jax.jit
def kernel(x, emb):
    diff = emb[:, 1] - emb[:, 0]
    p0, p1 = _sc_kernel(x.T, diff)
    return jnp.stack([p0, p1], axis=1)
